# baseline (speedup 1.0000x reference)
import functools

import jax
import jax.numpy as jnp
from jax import lax
from jax.experimental import pallas as pl
from jax.experimental.pallas import tpu as pltpu
from jax.experimental.pallas import tpu_sc as plsc

B = 16384
NW = 21
NUM_CORES = 2
NUM_SUBCORES = 16
L = 16
NWORK = NUM_CORES * NUM_SUBCORES
ROWS = B // NWORK
GROUPS = ROWS // L

TC_BLOCK = 1024


@functools.partial(
    pl.kernel,
    mesh=plsc.VectorSubcoreMesh(core_axis_name="c", subcore_axis_name="s"),
    compiler_params=pltpu.CompilerParams(needs_layout_passes=False),
    out_type=[
        jax.ShapeDtypeStruct((B * NW,), jnp.float32),
        jax.ShapeDtypeStruct((B,), jnp.int32),
    ],
    scratch_types=[
        pltpu.VMEM((ROWS * NW,), jnp.float32),
        pltpu.VMEM((ROWS * NW,), jnp.float32),
        pltpu.VMEM((ROWS,), jnp.int32),
    ],
)
def _sc_argmax(q_hbm, valid_hbm, idx_hbm, q_v, valid_v, idx_v):
    wid = lax.axis_index("s") * NUM_CORES + lax.axis_index("c")
    row0 = wid * ROWS
    pltpu.sync_copy(q_hbm.at[pl.ds(row0 * NW, ROWS * NW)], q_v)

    lanes = lax.iota(jnp.int32, L)

    def group(g, carry):
        r21 = (lanes + g * L) * NW
        best_v = plsc.load_gather(q_v, [r21])
        best_w = jnp.zeros((L,), jnp.int32)
        for w in range(1, NW):
            v = plsc.load_gather(q_v, [r21 + w])
            gt = v > best_v
            best_v = jnp.where(gt, v, best_v)
            best_w = jnp.where(gt, w, best_w)
        for w in range(NW):
            val = jnp.where(best_w == w, 1.0, 0.0).astype(jnp.float32)
            plsc.store_scatter(valid_v, [r21 + w], val)
        idx_v[pl.ds(g * L, L)] = best_w
        return carry

    lax.fori_loop(0, GROUPS, group, 0)

    pltpu.sync_copy(valid_v, valid_hbm.at[pl.ds(row0 * NW, ROWS * NW)])
    pltpu.sync_copy(idx_v, idx_hbm.at[pl.ds(row0, ROWS)])


def _tc_select_body(p_ref, v_ref, high_ref, low_ref):
    p = p_ref[...]
    v = v_ref[...]
    high_ref[...] = jnp.sum(p[:, :, 0] * v, axis=1, keepdims=True)
    low_ref[...] = jnp.sum(p[:, :, 1] * v, axis=1, keepdims=True)


_tc_select = pl.pallas_call(
    _tc_select_body,
    grid=(B // TC_BLOCK,),
    in_specs=[
        pl.BlockSpec((TC_BLOCK, NW, 2), lambda i: (i, 0, 0)),
        pl.BlockSpec((TC_BLOCK, NW), lambda i: (i, 0)),
    ],
    out_specs=[
        pl.BlockSpec((TC_BLOCK, 1), lambda i: (i, 0)),
        pl.BlockSpec((TC_BLOCK, 1), lambda i: (i, 0)),
    ],
    out_shape=[
        jax.ShapeDtypeStruct((B, 1), jnp.float32),
        jax.ShapeDtypeStruct((B, 1), jnp.float32),
    ],
)


def kernel(hidden_state, projections, quality_scores, r_squared,
           complete_cycles, position):
    del hidden_state, r_squared, complete_cycles, position
    q_flat = quality_scores.reshape(B * NW)
    valid_flat, idx = _sc_argmax(q_flat)
    valid = valid_flat.reshape(B, NW)
    high, low = _tc_select(projections, valid)
    return (high, low, valid, idx)


# speedup vs baseline: 3.4890x; 3.4890x over previous
import functools

import jax
import jax.numpy as jnp
from jax import lax
from jax.experimental import pallas as pl
from jax.experimental.pallas import tpu as pltpu
from jax.experimental.pallas import tpu_sc as plsc

B = 16384
NW = 21
NUM_CORES = 2
NUM_SUBCORES = 16
L = 16
NWORK = NUM_CORES * NUM_SUBCORES
ROWS = B // NWORK
GROUPS = ROWS // L

TC_BLOCK = 2048


@functools.partial(
    pl.kernel,
    mesh=plsc.VectorSubcoreMesh(core_axis_name="c", subcore_axis_name="s"),
    compiler_params=pltpu.CompilerParams(needs_layout_passes=False),
    out_type=[
        jax.ShapeDtypeStruct((B * NW,), jnp.float32),
        jax.ShapeDtypeStruct((B,), jnp.int32),
    ],
    scratch_types=[
        pltpu.VMEM((ROWS * NW,), jnp.float32),
        pltpu.VMEM((ROWS * NW,), jnp.float32),
        pltpu.VMEM((ROWS,), jnp.int32),
    ],
)
def _sc_argmax(q_hbm, valid_hbm, idx_hbm, q_v, valid_v, idx_v):
    wid = lax.axis_index("s") * NUM_CORES + lax.axis_index("c")
    row0 = wid * ROWS
    pltpu.sync_copy(q_hbm.at[pl.ds(row0 * NW, ROWS * NW)], q_v)

    lanes = lax.iota(jnp.int32, L)

    def group(g, carry):
        r21 = (lanes + g * L) * NW
        best_v = plsc.load_gather(q_v, [r21])
        best_w = jnp.zeros((L,), jnp.int32)
        for w in range(1, NW):
            v = plsc.load_gather(q_v, [r21 + w])
            gt = v > best_v
            best_v = jnp.where(gt, v, best_v)
            best_w = jnp.where(gt, w, best_w)
        for w in range(NW):
            val = jnp.where(best_w == w, 1.0, 0.0).astype(jnp.float32)
            plsc.store_scatter(valid_v, [r21 + w], val)
        idx_v[pl.ds(g * L, L)] = best_w
        return carry

    lax.fori_loop(0, GROUPS, group, 0)

    pltpu.sync_copy(valid_v, valid_hbm.at[pl.ds(row0 * NW, ROWS * NW)])
    pltpu.sync_copy(idx_v, idx_hbm.at[pl.ds(row0, ROWS)])


def _tc_select_body(ph_ref, plo_ref, v_ref, high_ref, low_ref):
    v = v_ref[...]
    high_ref[...] = jnp.sum(ph_ref[...] * v, axis=1, keepdims=True)
    low_ref[...] = jnp.sum(plo_ref[...] * v, axis=1, keepdims=True)


_tc_select = pl.pallas_call(
    _tc_select_body,
    grid=(B // TC_BLOCK,),
    in_specs=[
        pl.BlockSpec((TC_BLOCK, NW), lambda i: (i, 0)),
        pl.BlockSpec((TC_BLOCK, NW), lambda i: (i, 0)),
        pl.BlockSpec((TC_BLOCK, NW), lambda i: (i, 0)),
    ],
    out_specs=[
        pl.BlockSpec((TC_BLOCK, 1), lambda i: (i, 0)),
        pl.BlockSpec((TC_BLOCK, 1), lambda i: (i, 0)),
    ],
    out_shape=[
        jax.ShapeDtypeStruct((B, 1), jnp.float32),
        jax.ShapeDtypeStruct((B, 1), jnp.float32),
    ],
)


def kernel(hidden_state, projections, quality_scores, r_squared,
           complete_cycles, position):
    del hidden_state, r_squared, complete_cycles, position
    q_flat = quality_scores.reshape(B * NW)
    valid_flat, idx = _sc_argmax(q_flat)
    valid = valid_flat.reshape(B, NW)
    ph = projections[:, :, 0]
    plo = projections[:, :, 1]
    high, low = _tc_select(ph, plo, valid)
    return (high, low, valid, idx)


# speedup vs baseline: 4.0734x; 1.1675x over previous
import functools

import jax
import jax.numpy as jnp
from jax import lax
from jax.experimental import pallas as pl
from jax.experimental.pallas import tpu as pltpu
from jax.experimental.pallas import tpu_sc as plsc

B = 16384
NW = 21
NUM_CORES = 2
NUM_SUBCORES = 16
L = 16
NWORK = NUM_CORES * NUM_SUBCORES
ROWS = B // NWORK
GROUPS = ROWS // L
CHUNK = ROWS * NW


@functools.partial(
    pl.kernel,
    mesh=plsc.VectorSubcoreMesh(core_axis_name="c", subcore_axis_name="s"),
    compiler_params=pltpu.CompilerParams(needs_layout_passes=False),
    out_type=[
        jax.ShapeDtypeStruct((B,), jnp.float32),
        jax.ShapeDtypeStruct((B,), jnp.float32),
        jax.ShapeDtypeStruct((B * NW,), jnp.float32),
        jax.ShapeDtypeStruct((B,), jnp.int32),
    ],
    scratch_types=[
        pltpu.VMEM((CHUNK,), jnp.float32),
        pltpu.VMEM((CHUNK,), jnp.float32),
        pltpu.VMEM((CHUNK,), jnp.float32),
        pltpu.VMEM((CHUNK,), jnp.float32),
        pltpu.VMEM((ROWS,), jnp.float32),
        pltpu.VMEM((ROWS,), jnp.float32),
        pltpu.VMEM((ROWS,), jnp.int32),
    ],
)
def _sc_extract(q_hbm, p2_hbm, high_hbm, low_hbm, valid_hbm, idx_hbm,
                q_v, ph_v, plo_v, valid_v, high_v, low_v, idx_v):
    wid = lax.axis_index("s") * NUM_CORES + lax.axis_index("c")
    row0 = wid * ROWS
    base = row0 * NW
    pltpu.sync_copy(q_hbm.at[pl.ds(base, CHUNK)], q_v)
    pltpu.sync_copy(p2_hbm.at[pl.ds(base, CHUNK)], ph_v)
    pltpu.sync_copy(p2_hbm.at[pl.ds(B * NW + base, CHUNK)], plo_v)

    lanes = lax.iota(jnp.int32, L)

    def group(g, carry):
        r21 = (lanes + g * L) * NW
        best_v = plsc.load_gather(q_v, [r21])
        best_w = jnp.zeros((L,), jnp.int32)
        for w in range(1, NW):
            v = plsc.load_gather(q_v, [r21 + w])
            gt = v > best_v
            best_v = jnp.where(gt, v, best_v)
            best_w = jnp.where(gt, w, best_w)
        for w in range(NW):
            val = jnp.where(best_w == w, 1.0, 0.0).astype(jnp.float32)
            plsc.store_scatter(valid_v, [r21 + w], val)
        sel = r21 + best_w
        high_v[pl.ds(g * L, L)] = plsc.load_gather(ph_v, [sel])
        low_v[pl.ds(g * L, L)] = plsc.load_gather(plo_v, [sel])
        idx_v[pl.ds(g * L, L)] = best_w
        return carry

    lax.fori_loop(0, GROUPS, group, 0)

    pltpu.sync_copy(valid_v, valid_hbm.at[pl.ds(base, CHUNK)])
    pltpu.sync_copy(high_v, high_hbm.at[pl.ds(row0, ROWS)])
    pltpu.sync_copy(low_v, low_hbm.at[pl.ds(row0, ROWS)])
    pltpu.sync_copy(idx_v, idx_hbm.at[pl.ds(row0, ROWS)])


def kernel(hidden_state, projections, quality_scores, r_squared,
           complete_cycles, position):
    del hidden_state, r_squared, complete_cycles, position
    q_flat = quality_scores.reshape(B * NW)
    p2_flat = projections.transpose(2, 0, 1).reshape(2 * B * NW)
    high, low, valid_flat, idx = _sc_extract(q_flat, p2_flat)
    return (high[:, None], low[:, None], valid_flat.reshape(B, NW), idx)


# speedup vs baseline: 11.6131x; 2.8509x over previous
import functools

import jax
import jax.numpy as jnp
from jax import lax
from jax.experimental import pallas as pl
from jax.experimental.pallas import tpu as pltpu
from jax.experimental.pallas import tpu_sc as plsc

B = 16384
NW = 21
NUM_CORES = 2
NUM_SUBCORES = 16
L = 16
NWORK = NUM_CORES * NUM_SUBCORES
ROWS = B // NWORK
GROUPS = ROWS // L


@functools.partial(
    pl.kernel,
    mesh=plsc.VectorSubcoreMesh(core_axis_name="c", subcore_axis_name="s"),
    compiler_params=pltpu.CompilerParams(needs_layout_passes=False),
    out_type=[
        jax.ShapeDtypeStruct((B,), jnp.float32),
        jax.ShapeDtypeStruct((B,), jnp.float32),
        jax.ShapeDtypeStruct((NW, B), jnp.float32),
        jax.ShapeDtypeStruct((B,), jnp.int32),
    ],
    scratch_types=[
        pltpu.VMEM((NW, ROWS), jnp.float32),
        pltpu.VMEM((NW * 2, ROWS), jnp.float32),
        pltpu.VMEM((NW, ROWS), jnp.float32),
        pltpu.VMEM((ROWS,), jnp.float32),
        pltpu.VMEM((ROWS,), jnp.float32),
        pltpu.VMEM((ROWS,), jnp.int32),
    ],
)
def _sc_extract(qT_hbm, pT_hbm, high_hbm, low_hbm, validT_hbm, idx_hbm,
                q_v, p_v, valid_v, high_v, low_v, idx_v):
    wid = lax.axis_index("s") * NUM_CORES + lax.axis_index("c")
    row0 = wid * ROWS
    pltpu.sync_copy(qT_hbm.at[:, pl.ds(row0, ROWS)], q_v)
    pltpu.sync_copy(pT_hbm.at[:, pl.ds(row0, ROWS)], p_v)

    lanes = lax.iota(jnp.int32, L)

    def group(g, carry):
        col = g * L
        best_v = q_v[0, pl.ds(col, L)]
        best_w = jnp.zeros((L,), jnp.int32)
        for w in range(1, NW):
            v = q_v[w, pl.ds(col, L)]
            gt = v > best_v
            best_v = jnp.where(gt, v, best_v)
            best_w = jnp.where(gt, w, best_w)
        for w in range(NW):
            valid_v[w, pl.ds(col, L)] = jnp.where(
                best_w == w, 1.0, 0.0).astype(jnp.float32)
        cols = lanes + col
        high_v[pl.ds(col, L)] = plsc.load_gather(p_v, [best_w * 2, cols])
        low_v[pl.ds(col, L)] = plsc.load_gather(p_v, [best_w * 2 + 1, cols])
        idx_v[pl.ds(col, L)] = best_w
        return carry

    lax.fori_loop(0, GROUPS, group, 0)

    pltpu.sync_copy(valid_v, validT_hbm.at[:, pl.ds(row0, ROWS)])
    pltpu.sync_copy(high_v, high_hbm.at[pl.ds(row0, ROWS)])
    pltpu.sync_copy(low_v, low_hbm.at[pl.ds(row0, ROWS)])
    pltpu.sync_copy(idx_v, idx_hbm.at[pl.ds(row0, ROWS)])


def kernel(hidden_state, projections, quality_scores, r_squared,
           complete_cycles, position):
    del hidden_state, r_squared, complete_cycles, position
    qT = quality_scores.T
    pT = projections.transpose(1, 2, 0).reshape(NW * 2, B)
    high, low, validT, idx = _sc_extract(qT, pT)
    return (high[:, None], low[:, None], validT.T, idx)


# speedup vs baseline: 11.9458x; 1.0287x over previous
import functools

import jax
import jax.numpy as jnp
from jax import lax
from jax.experimental import pallas as pl
from jax.experimental.pallas import tpu as pltpu
from jax.experimental.pallas import tpu_sc as plsc

B = 16384
NW = 21
NUM_CORES = 2
NUM_SUBCORES = 16
L = 16
NWORK = NUM_CORES * NUM_SUBCORES
ROWS = B // NWORK
GROUPS = ROWS // L


@functools.partial(
    pl.kernel,
    mesh=plsc.VectorSubcoreMesh(core_axis_name="c", subcore_axis_name="s"),
    compiler_params=pltpu.CompilerParams(needs_layout_passes=False),
    out_type=[
        jax.ShapeDtypeStruct((B,), jnp.float32),
        jax.ShapeDtypeStruct((B,), jnp.float32),
        jax.ShapeDtypeStruct((NW, B), jnp.float32),
        jax.ShapeDtypeStruct((B,), jnp.int32),
    ],
    scratch_types=[
        pltpu.VMEM((NW, ROWS), jnp.float32),
        pltpu.VMEM((NW * 2, ROWS), jnp.float32),
        pltpu.VMEM((NW, ROWS), jnp.float32),
        pltpu.VMEM((ROWS,), jnp.float32),
        pltpu.VMEM((ROWS,), jnp.float32),
        pltpu.VMEM((ROWS,), jnp.int32),
        pltpu.SemaphoreType.DMA,
        pltpu.SemaphoreType.DMA,
        pltpu.SemaphoreType.DMA,
        pltpu.SemaphoreType.DMA,
    ],
)
def _sc_extract(qT_hbm, pT_hbm, high_hbm, low_hbm, validT_hbm, idx_hbm,
                q_v, p_v, valid_v, high_v, low_v, idx_v,
                sem_q, sem_p, sem_vi, sem_hl):
    wid = lax.axis_index("s") * NUM_CORES + lax.axis_index("c")
    row0 = wid * ROWS
    cq = pltpu.async_copy(qT_hbm.at[:, pl.ds(row0, ROWS)], q_v, sem_q)
    cp = pltpu.async_copy(pT_hbm.at[:, pl.ds(row0, ROWS)], p_v, sem_p)
    cq.wait()

    best_ws = []
    for g in range(GROUPS):
        col = g * L
        best_v = q_v[0, pl.ds(col, L)]
        best_w = jnp.zeros((L,), jnp.int32)
        for w in range(1, NW):
            v = q_v[w, pl.ds(col, L)]
            gt = v > best_v
            best_v = jnp.where(gt, v, best_v)
            best_w = jnp.where(gt, w, best_w)
        for w in range(NW):
            valid_v[w, pl.ds(col, L)] = jnp.where(
                best_w == w, 1.0, 0.0).astype(jnp.float32)
        idx_v[pl.ds(col, L)] = best_w
        best_ws.append(best_w)

    cv = pltpu.async_copy(valid_v, validT_hbm.at[:, pl.ds(row0, ROWS)], sem_vi)
    ci = pltpu.async_copy(idx_v, idx_hbm.at[pl.ds(row0, ROWS)], sem_vi)
    cp.wait()

    lanes = lax.iota(jnp.int32, L)
    for g in range(GROUPS):
        col = g * L
        cols = lanes + col
        best_w = best_ws[g]
        high_v[pl.ds(col, L)] = plsc.load_gather(p_v, [best_w * 2, cols])
        low_v[pl.ds(col, L)] = plsc.load_gather(p_v, [best_w * 2 + 1, cols])

    ch = pltpu.async_copy(high_v, high_hbm.at[pl.ds(row0, ROWS)], sem_hl)
    cl = pltpu.async_copy(low_v, low_hbm.at[pl.ds(row0, ROWS)], sem_hl)
    cv.wait()
    ci.wait()
    ch.wait()
    cl.wait()


def kernel(hidden_state, projections, quality_scores, r_squared,
           complete_cycles, position):
    del hidden_state, r_squared, complete_cycles, position
    qT = quality_scores.T
    pT = projections.transpose(1, 2, 0).reshape(NW * 2, B)
    high, low, validT, idx = _sc_extract(qT, pT)
    return (high[:, None], low[:, None], validT.T, idx)
